# Initial kernel scaffold; baseline (speedup 1.0000x reference)
import jax
import jax.numpy as jnp
from jax.experimental import pallas as pl


def kernel(x, edge_index, W1, b1, W2, b2, Wc, bc):
    raise NotImplementedError("write your pallas kernel here")


# speedup vs baseline: 11.6864x; 11.6864x over previous
import jax
import jax.numpy as jnp
from jax import lax
from jax.experimental import pallas as pl
from jax.experimental.pallas import tpu as pltpu
from jax.experimental.pallas import tpu_sc as plsc

_N = 10000
_D = 128
_H = 128
_C = 64
_E = 320000

_NC = 2
_NS = 16
_K = 80
_EPT = _E // (_NC * _NS)
_CHUNKS = _EPT // _K
_RPT = 624
_TAIL0 = _RPT * _NS
_TAILN = _N - _TAIL0
_DW = 128

_mesh = plsc.VectorSubcoreMesh(core_axis_name="c", subcore_axis_name="s")


def _node_copy(sid, src, dst, src_base, dst_base):
    r0 = sid * _RPT
    pltpu.sync_copy(src.at[pl.ds(src_base + r0, _RPT)],
                    dst.at[pl.ds(dst_base + r0, _RPT)])

    @pl.when(sid == _NS - 1)
    def _():
        pltpu.sync_copy(src.at[pl.ds(src_base + _TAIL0, _TAILN)],
                        dst.at[pl.ds(dst_base + _TAIL0, _TAILN)])


def _deg_body(dst_hbm, z_hbm, ones_hbm, out_hbm, idx_v, ones_v, acc_sh):
    cid = lax.axis_index("c")
    sid = lax.axis_index("s")
    e0 = (cid * _NS + sid) * _EPT
    pltpu.sync_copy(ones_hbm, ones_v)
    _node_copy(sid, z_hbm, acc_sh, 0, 0)
    plsc.subcore_barrier()

    def body(c, carry):
        off = e0 + c * _K
        pltpu.sync_copy(dst_hbm.at[pl.ds(off, _K)], idx_v)
        pltpu.sync_copy(ones_v, acc_sh.at[idx_v], add=True)
        return carry

    lax.fori_loop(0, _CHUNKS, body, 0)
    plsc.subcore_barrier()
    _node_copy(sid, acc_sh, out_hbm, 0, cid * _N)


_deg_call = pl.kernel(
    _deg_body,
    out_type=jax.ShapeDtypeStruct((2 * _N, _DW), jnp.float32),
    mesh=_mesh,
    scratch_types=[
        pltpu.VMEM((_K,), jnp.int32),
        pltpu.VMEM((_K, _DW), jnp.float32),
        pltpu.VMEM_SHARED((_N, _DW), jnp.float32),
    ],
)


def _agg_body(h_hbm, z_hbm, src_hbm, dst_hbm, out_hbm,
              src_v, dst_v, rows_v, acc_sh, gsem):
    cid = lax.axis_index("c")
    sid = lax.axis_index("s")
    e0 = (cid * _NS + sid) * _EPT

    @pl.when(cid == 0)
    def _():
        _node_copy(sid, h_hbm, acc_sh, 0, 0)

    @pl.when(cid == 1)
    def _():
        _node_copy(sid, z_hbm, acc_sh, 0, 0)

    plsc.subcore_barrier()

    def body(c, carry):
        off = e0 + c * _K
        pltpu.sync_copy(src_hbm.at[pl.ds(off, _K)], src_v)
        pltpu.sync_copy(dst_hbm.at[pl.ds(off, _K)], dst_v)
        pltpu.async_copy(h_hbm.at[src_v], rows_v, gsem).wait()
        pltpu.sync_copy(rows_v, acc_sh.at[dst_v], add=True)
        return carry

    lax.fori_loop(0, _CHUNKS, body, 0)
    plsc.subcore_barrier()
    _node_copy(sid, acc_sh, out_hbm, 0, cid * _N)


_agg_call = pl.kernel(
    _agg_body,
    out_type=jax.ShapeDtypeStruct((2 * _N, _H), jnp.float32),
    mesh=_mesh,
    scratch_types=[
        pltpu.VMEM((_K,), jnp.int32),
        pltpu.VMEM((_K,), jnp.int32),
        pltpu.VMEM((_K, _H), jnp.float32),
        pltpu.VMEM_SHARED((_N, _H), jnp.float32),
        pltpu.SemaphoreType.DMA,
    ],
)

_R = 1000


def _rspec(w):
    return pl.BlockSpec((_R, w), lambda i: (i, 0))


def _fspec(r, c):
    return pl.BlockSpec((r, c), lambda i: (0, 0))


def _dinv_of(d0_ref, d1_ref):
    return lax.rsqrt(d0_ref[:, 0:1] + d1_ref[:, 0:1] + 1.0)


def _scale_body(d0_ref, d1_ref, x_ref, w_ref, o_ref):
    dinv = _dinv_of(d0_ref, d1_ref)
    m = jnp.dot(x_ref[:], w_ref[:], preferred_element_type=jnp.float32,
                precision=lax.Precision.HIGHEST)
    o_ref[:] = m * dinv


_scale_call = pl.pallas_call(
    _scale_body,
    grid=(_N // _R,),
    in_specs=[_rspec(_DW), _rspec(_DW), _rspec(_D), _fspec(_D, _H)],
    out_specs=_rspec(_H),
    out_shape=jax.ShapeDtypeStruct((_N, _H), jnp.float32),
)


def _mid_body(s0_ref, s1_ref, d0_ref, d1_ref, b1_ref, w2_ref, o_ref):
    dinv = _dinv_of(d0_ref, d1_ref)
    h1 = jnp.maximum((s0_ref[:] + s1_ref[:]) * dinv + b1_ref[:], 0.0)
    m = jnp.dot(h1, w2_ref[:], preferred_element_type=jnp.float32,
                precision=lax.Precision.HIGHEST)
    o_ref[:] = m * dinv


_mid_call = pl.pallas_call(
    _mid_body,
    grid=(_N // _R,),
    in_specs=[_rspec(_H), _rspec(_H), _rspec(_DW), _rspec(_DW),
              _fspec(1, _H), _fspec(_H, _H)],
    out_specs=_rspec(_H),
    out_shape=jax.ShapeDtypeStruct((_N, _H), jnp.float32),
)


def _fin_body(t0_ref, t1_ref, d0_ref, d1_ref, b2_ref, wc_ref, bc_ref, o_ref):
    dinv = _dinv_of(d0_ref, d1_ref)
    h2 = (t0_ref[:] + t1_ref[:]) * dinv + b2_ref[:]
    logits = jnp.dot(h2, wc_ref[:], preferred_element_type=jnp.float32,
                     precision=lax.Precision.HIGHEST) + bc_ref[:]
    m = jnp.max(logits, axis=1, keepdims=True)
    lse = jnp.log(jnp.sum(jnp.exp(logits - m), axis=1, keepdims=True)) + m
    o_ref[:] = logits - lse


_fin_call = pl.pallas_call(
    _fin_body,
    grid=(_N // _R,),
    in_specs=[_rspec(_H), _rspec(_H), _rspec(_DW), _rspec(_DW),
              _fspec(1, _H), _fspec(_H, _C), _fspec(1, _C)],
    out_specs=_rspec(_C),
    out_shape=jax.ShapeDtypeStruct((_N, _C), jnp.float32),
)


def kernel(x, edge_index, W1, b1, W2, b2, Wc, bc):
    src = edge_index[0]
    dst = edge_index[1]
    z = jnp.zeros((_N, _H), jnp.float32)
    ones = jnp.ones((_K, _DW), jnp.float32)
    degs = _deg_call(dst, z, ones)
    d0, d1 = degs[:_N], degs[_N:]

    h1p = _scale_call(d0, d1, x, W1)
    s = _agg_call(h1p, z, src, dst)
    h2p = _mid_call(s[:_N], s[_N:], d0, d1, b1.reshape(1, _H), W2)
    t = _agg_call(h2p, z, src, dst)
    return _fin_call(t[:_N], t[_N:], d0, d1,
                     b2.reshape(1, _H), Wc, bc.reshape(1, _C))
